# baseline (speedup 1.0000x reference)
import functools

import jax
import jax.numpy as jnp
from jax import lax
from jax.experimental import pallas as pl
from jax.experimental.pallas import tpu as pltpu
from jax.experimental.pallas import tpu_sc as plsc

N = 1000000
FEATURES = 16
BATCH = 16384
P = FEATURES * 3
NC, NS, L = 2, 16, 16
NW = NC * NS
BPW = BATCH // NW
ICH = 128
NCH = BPW // ICH


def _sc_body(scale1, bias1, idx_hbm, sh48, out48,
             idx_v, idx2_v, s_v, b_v, sh_v, sem_g, sem_l):
    wid = lax.axis_index("s") * NC + lax.axis_index("c")
    base = wid * BPW
    pltpu.sync_copy(idx_hbm.at[pl.ds(base, BPW)], idx_v)
    sh_cp = pltpu.make_async_copy(sh48.at[:, pl.ds(base, BPW)], sh_v, sem_l)
    sh_cp.start()

    def plane(p, carry):
        off = p * N
        for k in range(BPW // L):
            s16 = pl.ds(k * L, L)
            idx2_v[s16] = idx_v[s16] + off
        cps = []
        for j in range(NCH):
            ich = idx2_v.at[pl.ds(j * ICH, ICH)]
            dst = pl.ds(j * ICH, ICH)
            cs = pltpu.make_async_copy(scale1.at[ich], s_v.at[p, dst], sem_g)
            cb = pltpu.make_async_copy(bias1.at[ich], b_v.at[p, dst], sem_g)
            cs.start()
            cb.start()
            cps.append(cs)
            cps.append(cb)
        for c in cps:
            c.wait()
        return carry

    lax.fori_loop(0, P, plane, 0)
    sh_cp.wait()

    def fma_plane(p, carry):
        for k in range(BPW // L):
            s16 = pl.ds(k * L, L)
            sh_v[p, s16] = sh_v[p, s16] * s_v[p, s16] + b_v[p, s16]
        return carry

    lax.fori_loop(0, P, fma_plane, 0)
    pltpu.sync_copy(sh_v, out48.at[:, pl.ds(base, BPW)])


@jax.jit
def kernel(idx, sh, scale, bias):
    scale1 = scale.transpose(2, 1, 0).reshape(P * N)
    bias1 = bias.transpose(2, 1, 0).reshape(P * N)
    sh48 = sh.transpose(2, 1, 0).reshape(P, BATCH)
    mesh = plsc.VectorSubcoreMesh(core_axis_name="c", subcore_axis_name="s")
    run = functools.partial(
        pl.kernel,
        mesh=mesh,
        out_type=jax.ShapeDtypeStruct((P, BATCH), jnp.float32),
        scratch_types=[
            pltpu.VMEM((BPW,), jnp.int32),
            pltpu.VMEM((BPW,), jnp.int32),
            pltpu.VMEM((P, BPW), jnp.float32),
            pltpu.VMEM((P, BPW), jnp.float32),
            pltpu.VMEM((P, BPW), jnp.float32),
            pltpu.SemaphoreType.DMA,
            pltpu.SemaphoreType.DMA,
        ],
    )(_sc_body)
    out48 = run(scale1, bias1, idx, sh48)
    return out48.reshape(3, FEATURES, BATCH).transpose(2, 1, 0)


# speedup vs baseline: 1.0039x; 1.0039x over previous
import functools

import jax
import jax.numpy as jnp
from jax import lax
from jax.experimental import pallas as pl
from jax.experimental.pallas import tpu as pltpu
from jax.experimental.pallas import tpu_sc as plsc

N = 1000000
FEATURES = 16
BATCH = 16384
P = FEATURES * 3
NC, NS, L = 2, 16, 16
NW = NC * NS
BPW = BATCH // NW
ICH = 128
NCH = BPW // ICH


def _sc_body(scale1, bias1, idx_hbm, sh48, out48,
             idx_v, idx2_v, s_v, b_v, sh_v, sem_g, sem_l):
    wid = lax.axis_index("s") * NC + lax.axis_index("c")
    base = wid * BPW
    pltpu.sync_copy(idx_hbm.at[pl.ds(base, BPW)], idx_v)
    sh_cp = pltpu.make_async_copy(sh48.at[:, pl.ds(base, BPW)], sh_v, sem_l)
    sh_cp.start()

    def expand(p, carry):
        off = p * N
        for k in range(BPW // L):
            s16 = pl.ds(k * L, L)
            idx2_v[p, s16] = idx_v[s16] + off
        return carry

    lax.fori_loop(0, P, expand, 0)

    def fire(p, carry):
        for j in range(NCH):
            ich = idx2_v.at[p, pl.ds(j * ICH, ICH)]
            dst = pl.ds(j * ICH, ICH)
            pltpu.make_async_copy(scale1.at[ich], s_v.at[p, dst], sem_g).start()
            pltpu.make_async_copy(bias1.at[ich], b_v.at[p, dst], sem_g).start()
        return carry

    lax.fori_loop(0, P, fire, 0)

    pltpu.make_async_copy(sh48.at[:, pl.ds(0, BPW)], s_v, sem_g).wait()
    pltpu.make_async_copy(sh48.at[:, pl.ds(0, BPW)], b_v, sem_g).wait()
    sh_cp.wait()

    def fma_plane(p, carry):
        for k in range(BPW // L):
            s16 = pl.ds(k * L, L)
            sh_v[p, s16] = sh_v[p, s16] * s_v[p, s16] + b_v[p, s16]
        return carry

    lax.fori_loop(0, P, fma_plane, 0)
    pltpu.sync_copy(sh_v, out48.at[:, pl.ds(base, BPW)])


@jax.jit
def kernel(idx, sh, scale, bias):
    scale1 = scale.transpose(2, 1, 0).reshape(P * N)
    bias1 = bias.transpose(2, 1, 0).reshape(P * N)
    sh48 = sh.transpose(2, 1, 0).reshape(P, BATCH)
    mesh = plsc.VectorSubcoreMesh(core_axis_name="c", subcore_axis_name="s")
    run = functools.partial(
        pl.kernel,
        mesh=mesh,
        out_type=jax.ShapeDtypeStruct((P, BATCH), jnp.float32),
        scratch_types=[
            pltpu.VMEM((BPW,), jnp.int32),
            pltpu.VMEM((P, BPW), jnp.int32),
            pltpu.VMEM((P, BPW), jnp.float32),
            pltpu.VMEM((P, BPW), jnp.float32),
            pltpu.VMEM((P, BPW), jnp.float32),
            pltpu.SemaphoreType.DMA,
            pltpu.SemaphoreType.DMA,
        ],
    )(_sc_body)
    out48 = run(scale1, bias1, idx, sh48)
    return out48.reshape(3, FEATURES, BATCH).transpose(2, 1, 0)


# speedup vs baseline: 2.7223x; 2.7118x over previous
import functools

import jax
import jax.numpy as jnp
from jax import lax
from jax.experimental import pallas as pl
from jax.experimental.pallas import tpu as pltpu
from jax.experimental.pallas import tpu_sc as plsc

N = 1000000
FEATURES = 16
BATCH = 16384
ROW = FEATURES * 3
NC, NS, L = 2, 16, 16
NW = NC * NS
BPW = BATCH // NW
ICHUNK = 128
NCHUNK = BPW // ICHUNK


def _sc_body(scale_hbm, bias_hbm, idx_hbm, sh_hbm, out_hbm,
             idx_v, s_v, b_v, sh_v, sem_g, sem_l):
    wid = lax.axis_index("s") * NC + lax.axis_index("c")
    base = wid * BPW

    pltpu.sync_copy(idx_hbm.at[wid], idx_v)

    sh_cp = pltpu.make_async_copy(sh_hbm.at[pl.ds(base, BPW)], sh_v, sem_l)
    sh_cp.start()
    copies = []
    for j in range(NCHUNK):
        dst = pl.ds(j * ICHUNK, ICHUNK)
        cs = pltpu.make_async_copy(scale_hbm.at[idx_v.at[j]], s_v.at[dst], sem_g)
        cb = pltpu.make_async_copy(bias_hbm.at[idx_v.at[j]], b_v.at[dst], sem_g)
        cs.start()
        cb.start()
        copies.append(cs)
        copies.append(cb)
    sh_cp.wait()
    for c in copies:
        c.wait()

    def fma_row(r, carry):
        for c in range(ROW // L):
            cols = pl.ds(c * L, L)
            sh_v[r, cols] = sh_v[r, cols] * s_v[r, cols] + b_v[r, cols]
        return carry

    lax.fori_loop(0, BPW, fma_row, 0, unroll=4)

    pltpu.sync_copy(sh_v, out_hbm.at[pl.ds(base, BPW)])


@jax.jit
def kernel(idx, sh, scale, bias):
    scale2 = scale.reshape(N, ROW)
    bias2 = bias.reshape(N, ROW)
    sh2 = sh.reshape(BATCH, ROW)
    idx3 = idx.reshape(NW, NCHUNK, ICHUNK)
    mesh = plsc.VectorSubcoreMesh(core_axis_name="c", subcore_axis_name="s")
    run = functools.partial(
        pl.kernel,
        mesh=mesh,
        compiler_params=pltpu.CompilerParams(use_tc_tiling_on_sc=False),
        out_type=jax.ShapeDtypeStruct((BATCH, ROW), jnp.float32),
        scratch_types=[
            pltpu.VMEM((NCHUNK, ICHUNK), jnp.int32),
            pltpu.VMEM((BPW, ROW), jnp.float32),
            pltpu.VMEM((BPW, ROW), jnp.float32),
            pltpu.VMEM((BPW, ROW), jnp.float32),
            pltpu.SemaphoreType.DMA,
            pltpu.SemaphoreType.DMA,
        ],
    )(_sc_body)
    out = run(scale2, bias2, idx3, sh2)
    return out.reshape(BATCH, FEATURES, 3)
